# Initial kernel scaffold; baseline (speedup 1.0000x reference)
import jax
import jax.numpy as jnp
from jax.experimental import pallas as pl


def kernel(adj, norm, h, weight, bias):
    raise NotImplementedError("write your pallas kernel here")


# speedup vs baseline: 1.3562x; 1.3562x over previous
import jax
import jax.numpy as jnp
from jax.experimental import pallas as pl
from jax.experimental.pallas import tpu as pltpu


def _round_up(x, m):
    return ((x + m - 1) // m) * m


def _pick_tile(n, target, align):
    if n <= target:
        return n
    best = None
    t = align
    while t <= target:
        if n % t == 0:
            best = t
        t += align
    return best if best is not None else n


def _transform_kernel(h_ref, w_ref, norm_ref, y_ref):
    xw = jnp.dot(h_ref[...], w_ref[...], preferred_element_type=jnp.float32)
    y_ref[...] = (xw * norm_ref[...]).astype(jnp.bfloat16)


def _make_agg_kernel(nk, tk):
    def _agg_kernel(adj_ref, y_ref, norm_ref, bias_ref, out_ref, acc_ref):
        k = pl.program_id(1)
        a16 = adj_ref[...].astype(jnp.bfloat16)
        yk = y_ref[pl.ds(k * tk, tk), :]
        prod = jnp.dot(a16, yk, preferred_element_type=jnp.float32)

        @pl.when(k == 0)
        def _():
            acc_ref[...] = prod

        @pl.when(k != 0)
        def _():
            acc_ref[...] += prod

        @pl.when(k == nk - 1)
        def _():
            res = acc_ref[...] * norm_ref[...] + bias_ref[...]
            out_ref[...] = jnp.maximum(res, 0.0).astype(out_ref.dtype)

    return _agg_kernel


def kernel(adj, norm, h, weight, bias):
    N, F_in = h.shape
    F_out = weight.shape[1]

    F_pad = _round_up(max(F_out, 128), 128)
    if F_pad != F_out:
        w_pad = jnp.zeros((F_in, F_pad), weight.dtype).at[:, :F_out].set(weight)
        b_pad = jnp.zeros((1, F_pad), bias.dtype).at[0, :F_out].set(bias)
    else:
        w_pad = weight
        b_pad = bias.reshape(1, F_out)

    TM = _pick_tile(N, 512, 8)
    TK = _pick_tile(N, 1024, 128)
    NM = N // TM
    NK = N // TK

    y = pl.pallas_call(
        _transform_kernel,
        out_shape=jax.ShapeDtypeStruct((N, F_pad), jnp.bfloat16),
        grid_spec=pl.GridSpec(
            grid=(NM,),
            in_specs=[
                pl.BlockSpec((TM, F_in), lambda i: (i, 0)),
                pl.BlockSpec((F_in, F_pad), lambda i: (0, 0)),
                pl.BlockSpec((TM, 1), lambda i: (i, 0)),
            ],
            out_specs=pl.BlockSpec((TM, F_pad), lambda i: (i, 0)),
        ),
        compiler_params=pltpu.CompilerParams(
            dimension_semantics=("parallel",)),
    )(h, w_pad, norm)

    out = pl.pallas_call(
        _make_agg_kernel(NK, TK),
        out_shape=jax.ShapeDtypeStruct((N, F_pad), h.dtype),
        grid_spec=pltpu.PrefetchScalarGridSpec(
            num_scalar_prefetch=0,
            grid=(NM, NK),
            in_specs=[
                pl.BlockSpec((TM, TK), lambda i, k: (i, k)),
                pl.BlockSpec((N, F_pad), lambda i, k: (0, 0)),
                pl.BlockSpec((TM, 1), lambda i, k: (i, 0)),
                pl.BlockSpec((1, F_pad), lambda i, k: (0, 0)),
            ],
            out_specs=pl.BlockSpec((TM, F_pad), lambda i, k: (i, 0)),
            scratch_shapes=[pltpu.VMEM((TM, F_pad), jnp.float32)],
        ),
        compiler_params=pltpu.CompilerParams(
            dimension_semantics=("parallel", "arbitrary"),
            vmem_limit_bytes=48 << 20),
    )(adj, y, norm, b_pad)

    if F_pad != F_out:
        out = out[:, :F_out]
    return out


# speedup vs baseline: 1.9260x; 1.4201x over previous
import jax
import jax.numpy as jnp
from jax.experimental import pallas as pl
from jax.experimental.pallas import tpu as pltpu


def _round_up(x, m):
    return ((x + m - 1) // m) * m


def _pick_tile(n, target, align):
    if n <= target:
        return n
    best = None
    t = align
    while t <= target:
        if n % t == 0:
            best = t
        t += align
    return best if best is not None else n


def _transform_kernel(h_ref, w_ref, norm_ref, y_ref):
    xw = jnp.dot(h_ref[...], w_ref[...], preferred_element_type=jnp.float32)
    y_ref[...] = (xw * norm_ref[...]).astype(jnp.bfloat16)


def _agg_kernel_fullk(adj_ref, y_ref, norm_ref, bias_ref, out_ref):
    a16 = adj_ref[...].astype(jnp.bfloat16)
    acc = jnp.dot(a16, y_ref[...], preferred_element_type=jnp.float32)
    res = acc * norm_ref[...] + bias_ref[...]
    out_ref[...] = jnp.maximum(res, 0.0).astype(out_ref.dtype)


def _make_agg_kernel(nk, tk):
    def _agg_kernel(adj_ref, y_ref, norm_ref, bias_ref, out_ref, acc_ref):
        k = pl.program_id(1)
        a16 = adj_ref[...].astype(jnp.bfloat16)
        yk = y_ref[pl.ds(k * tk, tk), :]
        prod = jnp.dot(a16, yk, preferred_element_type=jnp.float32)

        @pl.when(k == 0)
        def _():
            acc_ref[...] = prod

        @pl.when(k != 0)
        def _():
            acc_ref[...] += prod

        @pl.when(k == nk - 1)
        def _():
            res = acc_ref[...] * norm_ref[...] + bias_ref[...]
            out_ref[...] = jnp.maximum(res, 0.0).astype(out_ref.dtype)

    return _agg_kernel


def kernel(adj, norm, h, weight, bias):
    N, F_in = h.shape
    F_out = weight.shape[1]

    F_pad = _round_up(max(F_out, 128), 128)
    if F_pad != F_out:
        w_pad = jnp.zeros((F_in, F_pad), weight.dtype).at[:, :F_out].set(weight)
        b_pad = jnp.zeros((1, F_pad), bias.dtype).at[0, :F_out].set(bias)
    else:
        w_pad = weight
        b_pad = bias.reshape(1, F_out)

    TM = _pick_tile(N, 512, 8)
    TK = _pick_tile(N, 4096, 128)
    NM = N // TM
    NK = N // TK

    y = pl.pallas_call(
        _transform_kernel,
        out_shape=jax.ShapeDtypeStruct((N, F_pad), jnp.bfloat16),
        grid_spec=pl.GridSpec(
            grid=(NM,),
            in_specs=[
                pl.BlockSpec((TM, F_in), lambda i: (i, 0)),
                pl.BlockSpec((F_in, F_pad), lambda i: (0, 0)),
                pl.BlockSpec((TM, 1), lambda i: (i, 0)),
            ],
            out_specs=pl.BlockSpec((TM, F_pad), lambda i: (i, 0)),
        ),
        compiler_params=pltpu.CompilerParams(
            dimension_semantics=("parallel",)),
    )(h, w_pad, norm)

    if NK == 1:
        out = pl.pallas_call(
            _agg_kernel_fullk,
            out_shape=jax.ShapeDtypeStruct((N, F_pad), h.dtype),
            grid_spec=pl.GridSpec(
                grid=(NM,),
                in_specs=[
                    pl.BlockSpec((TM, N), lambda i: (i, 0)),
                    pl.BlockSpec((N, F_pad), lambda i: (0, 0)),
                    pl.BlockSpec((TM, 1), lambda i: (i, 0)),
                    pl.BlockSpec((1, F_pad), lambda i: (0, 0)),
                ],
                out_specs=pl.BlockSpec((TM, F_pad), lambda i: (i, 0)),
            ),
            compiler_params=pltpu.CompilerParams(
                dimension_semantics=("arbitrary",),
                vmem_limit_bytes=48 << 20),
        )(adj, y, norm, b_pad)
    else:
        out = pl.pallas_call(
            _make_agg_kernel(NK, TK),
            out_shape=jax.ShapeDtypeStruct((N, F_pad), h.dtype),
            grid_spec=pltpu.PrefetchScalarGridSpec(
                num_scalar_prefetch=0,
                grid=(NM, NK),
                in_specs=[
                    pl.BlockSpec((TM, TK), lambda i, k: (i, k)),
                    pl.BlockSpec((N, F_pad), lambda i, k: (0, 0)),
                    pl.BlockSpec((TM, 1), lambda i, k: (i, 0)),
                    pl.BlockSpec((1, F_pad), lambda i, k: (0, 0)),
                ],
                out_specs=pl.BlockSpec((TM, F_pad), lambda i, k: (i, 0)),
                scratch_shapes=[pltpu.VMEM((TM, F_pad), jnp.float32)],
            ),
            compiler_params=pltpu.CompilerParams(
                dimension_semantics=("parallel", "arbitrary"),
                vmem_limit_bytes=48 << 20),
        )(adj, y, norm, b_pad)

    if F_pad != F_out:
        out = out[:, :F_out]
    return out


# speedup vs baseline: 1.9329x; 1.0036x over previous
import jax
import jax.numpy as jnp
from jax.experimental import pallas as pl
from jax.experimental.pallas import tpu as pltpu


def _round_up(x, m):
    return ((x + m - 1) // m) * m


def _pick_tile(n, target, align):
    if n <= target:
        return n
    best = None
    t = align
    while t <= target:
        if n % t == 0:
            best = t
        t += align
    return best if best is not None else n


def _transform_kernel(h_ref, w_ref, norm_ref, y_ref):
    xw = jnp.dot(h_ref[...], w_ref[...], preferred_element_type=jnp.float32)
    y_ref[...] = (xw * norm_ref[...]).astype(jnp.bfloat16)


def _agg_kernel_fullk(adj_ref, y_ref, norm_ref, bias_ref, out_ref):
    a16 = adj_ref[...].astype(jnp.bfloat16)
    acc = jnp.dot(a16, y_ref[...], preferred_element_type=jnp.float32)
    res = acc * norm_ref[...] + bias_ref[...]
    out_ref[...] = jnp.maximum(res, 0.0).astype(out_ref.dtype)


def _make_agg_kernel(nk, tk):
    def _agg_kernel(adj_ref, y_ref, norm_ref, bias_ref, out_ref, acc_ref):
        k = pl.program_id(1)
        a16 = adj_ref[...].astype(jnp.bfloat16)
        yk = y_ref[pl.ds(k * tk, tk), :]
        prod = jnp.dot(a16, yk, preferred_element_type=jnp.float32)

        @pl.when(k == 0)
        def _():
            acc_ref[...] = prod

        @pl.when(k != 0)
        def _():
            acc_ref[...] += prod

        @pl.when(k == nk - 1)
        def _():
            res = acc_ref[...] * norm_ref[...] + bias_ref[...]
            out_ref[...] = jnp.maximum(res, 0.0).astype(out_ref.dtype)

    return _agg_kernel


def kernel(adj, norm, h, weight, bias):
    N, F_in = h.shape
    F_out = weight.shape[1]

    F_pad = _round_up(max(F_out, 128), 128)
    if F_pad != F_out:
        w_pad = jnp.zeros((F_in, F_pad), weight.dtype).at[:, :F_out].set(weight)
        b_pad = jnp.zeros((1, F_pad), bias.dtype).at[0, :F_out].set(bias)
    else:
        w_pad = weight
        b_pad = bias.reshape(1, F_out)

    TM = _pick_tile(N, 1024, 8)
    TK = _pick_tile(N, 4096, 128)
    NM = N // TM
    NK = N // TK

    y = pl.pallas_call(
        _transform_kernel,
        out_shape=jax.ShapeDtypeStruct((N, F_pad), jnp.bfloat16),
        grid_spec=pl.GridSpec(
            grid=(NM,),
            in_specs=[
                pl.BlockSpec((TM, F_in), lambda i: (i, 0)),
                pl.BlockSpec((F_in, F_pad), lambda i: (0, 0)),
                pl.BlockSpec((TM, 1), lambda i: (i, 0)),
            ],
            out_specs=pl.BlockSpec((TM, F_pad), lambda i: (i, 0)),
        ),
        compiler_params=pltpu.CompilerParams(
            dimension_semantics=("parallel",)),
    )(h, w_pad, norm)

    if NK == 1:
        out = pl.pallas_call(
            _agg_kernel_fullk,
            out_shape=jax.ShapeDtypeStruct((N, F_pad), h.dtype),
            grid_spec=pl.GridSpec(
                grid=(NM,),
                in_specs=[
                    pl.BlockSpec((TM, N), lambda i: (i, 0)),
                    pl.BlockSpec((N, F_pad), lambda i: (0, 0)),
                    pl.BlockSpec((TM, 1), lambda i: (i, 0)),
                    pl.BlockSpec((1, F_pad), lambda i: (0, 0)),
                ],
                out_specs=pl.BlockSpec((TM, F_pad), lambda i: (i, 0)),
            ),
            compiler_params=pltpu.CompilerParams(
                dimension_semantics=("arbitrary",),
                vmem_limit_bytes=48 << 20),
        )(adj, y, norm, b_pad)
    else:
        out = pl.pallas_call(
            _make_agg_kernel(NK, TK),
            out_shape=jax.ShapeDtypeStruct((N, F_pad), h.dtype),
            grid_spec=pltpu.PrefetchScalarGridSpec(
                num_scalar_prefetch=0,
                grid=(NM, NK),
                in_specs=[
                    pl.BlockSpec((TM, TK), lambda i, k: (i, k)),
                    pl.BlockSpec((N, F_pad), lambda i, k: (0, 0)),
                    pl.BlockSpec((TM, 1), lambda i, k: (i, 0)),
                    pl.BlockSpec((1, F_pad), lambda i, k: (0, 0)),
                ],
                out_specs=pl.BlockSpec((TM, F_pad), lambda i, k: (i, 0)),
                scratch_shapes=[pltpu.VMEM((TM, F_pad), jnp.float32)],
            ),
            compiler_params=pltpu.CompilerParams(
                dimension_semantics=("parallel", "arbitrary"),
                vmem_limit_bytes=48 << 20),
        )(adj, y, norm, b_pad)

    if F_pad != F_out:
        out = out[:, :F_out]
    return out
